# Initial kernel scaffold; baseline (speedup 1.0000x reference)
import jax
import jax.numpy as jnp
from jax.experimental import pallas as pl


def kernel(feat, edge_index, edge_weight, ntype_idxs, etype_idxs, W, A_l, A_r, W_res, b_res):
    raise NotImplementedError("write your pallas kernel here")


# speedup vs baseline: 124.5086x; 124.5086x over previous
import functools

import jax
import jax.numpy as jnp
from jax import lax
from jax.experimental import pallas as pl
from jax.experimental.pallas import tpu as pltpu
from jax.experimental.pallas import tpu_sc as plsc

N = 10000
E = 320000
DIN = 128
H = 4
D = 8
HD = H * D
T = 5

NC = 2
NS = 16
NW = NC * NS
CH = 128
EPAD = 327680
CPW = EPAD // (NW * CH)
NPAD = 10240
ROWS_PT = NPAD // NS

_TC_BN = 2000


def _tc_body(feat_ref, wall_ref, bres_ref, p_ref, res_ref):
    x = feat_ref[...]
    y = jnp.dot(x, wall_ref[...], preferred_element_type=jnp.float32)
    for t in range(T):
        z = y[:, t * 64:t * 64 + HD]
        l = y[:, t * 64 + HD:t * 64 + 2 * HD]
        s = jax.nn.sigmoid(jnp.where(l >= 0, l, 0.2 * l))
        p_ref[t, :, :] = z * s
    res_ref[...] = (y[:, T * 64:T * 64 + HD] + bres_ref[...]) * 0.5


def _tc_precompute(feat, wall, bres):
    grid = (N // _TC_BN,)
    return pl.pallas_call(
        _tc_body,
        grid=grid,
        in_specs=[
            pl.BlockSpec((_TC_BN, DIN), lambda i: (i, 0)),
            pl.BlockSpec((DIN, T * 64 + HD), lambda i: (0, 0)),
            pl.BlockSpec((1, HD), lambda i: (0, 0)),
        ],
        out_specs=[
            pl.BlockSpec((T, _TC_BN, HD), lambda i: (0, i, 0)),
            pl.BlockSpec((_TC_BN, HD), lambda i: (i, 0)),
        ],
        out_shape=[
            jax.ShapeDtypeStruct((T, N, HD), jnp.float32),
            jax.ShapeDtypeStruct((N, HD), jnp.float32),
        ],
        compiler_params=pltpu.CompilerParams(
            dimension_semantics=("parallel",)),
    )(feat, wall, bres)


def _sc_body(p_hbm, resh_hbm, src_hbm, et_hbm, dst_hbm, ew_hbm, out_hbm,
             src_v, et_v, gidx_v, dst_v, ew_v, rows_v, acc, gsem):
    c = lax.axis_index("c")
    s = lax.axis_index("s")
    w = s * NC + c
    row0 = w * CPW

    pltpu.sync_copy(src_hbm.at[pl.ds(row0, CPW)], src_v)
    pltpu.sync_copy(et_hbm.at[pl.ds(row0, CPW)], et_v)
    pltpu.sync_copy(dst_hbm.at[pl.ds(row0, CPW)], dst_v)
    pltpu.sync_copy(ew_hbm.at[pl.ds(row0 * CH, CPW * CH)], ew_v)

    pltpu.sync_copy(resh_hbm.at[pl.ds(s * ROWS_PT, ROWS_PT)],
                    acc.at[pl.ds(s * ROWS_PT, ROWS_PT)])

    def _gidx_row(r, _):
        for h in range(CH // 16):
            sl = pl.ds(h * 16, 16)
            gidx_v[r, sl] = et_v[r, sl] * N + src_v[r, sl]
        return 0

    lax.fori_loop(0, CPW, _gidx_row, 0)

    plsc.subcore_barrier()

    zeros16 = jnp.zeros((16,), jnp.int32)

    def _chunk(j, _):
        pltpu.async_copy(p_hbm.at[gidx_v.at[j]], rows_v, gsem).wait()

        def _group(g, _):
            ew16 = ew_v[pl.ds(j * CH + g * 16, 16)]
            for k16 in range(16):
                k = g * 16 + k16
                wsp = jnp.zeros((16,), jnp.float32) + ew16[k16]
                rows_v[k, pl.ds(0, 16)] = rows_v[k, pl.ds(0, 16)] * wsp
                rows_v[k, pl.ds(16, 16)] = rows_v[k, pl.ds(16, 16)] * wsp
            return 0

        lax.fori_loop(0, CH // 16, _group, 0)

        pltpu.sync_copy(rows_v, acc.at[dst_v.at[j]], add=True)
        return 0

    lax.fori_loop(0, CPW, _chunk, 0)

    plsc.subcore_barrier()

    pltpu.sync_copy(acc.at[pl.ds(s * ROWS_PT, ROWS_PT)],
                    out_hbm.at[c, pl.ds(s * ROWS_PT, ROWS_PT)])


@functools.cache
def _sc_scatter():
    return pl.kernel(
        _sc_body,
        out_type=jax.ShapeDtypeStruct((NC, NPAD, HD), jnp.float32),
        mesh=plsc.VectorSubcoreMesh(core_axis_name="c", subcore_axis_name="s"),
        scratch_types=[
            pltpu.VMEM((CPW, CH), jnp.int32),
            pltpu.VMEM((CPW, CH), jnp.int32),
            pltpu.VMEM((CPW, CH), jnp.int32),
            pltpu.VMEM((CPW, CH), jnp.int32),
            pltpu.VMEM((CPW * CH,), jnp.float32),
            pltpu.VMEM((CH, HD), jnp.float32),
            pltpu.VMEM_SHARED((NPAD, HD), jnp.float32),
            pltpu.SemaphoreType.DMA,
        ],
        compiler_params=pltpu.CompilerParams(use_tc_tiling_on_sc=False),
    )


def kernel(feat, edge_index, edge_weight, ntype_idxs, etype_idxs,
           W, A_l, A_r, W_res, b_res):
    B = (A_l + A_r).reshape(T, HD, H, D).sum(-1)
    C8 = jnp.tile(jnp.einsum('tik,tkj->tij', W, B), (1, 1, D))
    Wp = W.reshape(T, DIN, H, D).transpose(0, 1, 3, 2).reshape(T, DIN, HD)
    wall = jnp.concatenate(
        [jnp.concatenate([Wp[t], C8[t]], axis=1) for t in range(T)]
        + [W_res], axis=1)

    p_tab, res_half = _tc_precompute(feat, wall, b_res.reshape(1, HD))
    p_flat = p_tab.reshape(T * N, HD)
    res_half = jnp.pad(res_half, ((0, NPAD - N), (0, 0)))

    src = edge_index[0].astype(jnp.int32)
    dst = edge_index[1].astype(jnp.int32)
    et = etype_idxs.astype(jnp.int32)
    ew = edge_weight.astype(jnp.float32)

    pad = EPAD - E
    srcm = jnp.concatenate([src, jnp.zeros((pad,), jnp.int32)]).reshape(-1, CH)
    dstm = jnp.concatenate([dst, jnp.zeros((pad,), jnp.int32)]).reshape(-1, CH)
    etm = jnp.concatenate([et, jnp.zeros((pad,), jnp.int32)]).reshape(-1, CH)
    ewm = jnp.concatenate([ew, jnp.zeros((pad,), jnp.float32)])

    parts = _sc_scatter()(p_flat, res_half, srcm, etm, dstm, ewm)
    return parts[0, :N] + parts[1, :N]


# speedup vs baseline: 157.9649x; 1.2687x over previous
import functools

import jax
import jax.numpy as jnp
from jax import lax
from jax.experimental import pallas as pl
from jax.experimental.pallas import tpu as pltpu
from jax.experimental.pallas import tpu_sc as plsc

N = 10000
E = 320000
DIN = 128
H = 4
D = 8
HD = H * D
T = 5

NC = 2
NS = 16
NW = NC * NS
CH = 128
EPAD = 327680
CPW = EPAD // (NW * CH)
NPAD = 10240
ROWS_PT = NPAD // NS

_TC_BN = 2000


def _tc_body(feat_ref, wall_ref, bres_ref, p_ref, res_ref):
    x = feat_ref[...]
    y = jnp.dot(x, wall_ref[...], preferred_element_type=jnp.float32)
    for t in range(T):
        z = y[:, t * 64:t * 64 + HD]
        l = y[:, t * 64 + HD:t * 64 + 2 * HD]
        s = jax.nn.sigmoid(jnp.where(l >= 0, l, 0.2 * l))
        p_ref[t, :, :] = z * s
    res_ref[...] = (y[:, T * 64:T * 64 + HD] + bres_ref[...]) * 0.5


def _tc_precompute(feat, wall, bres):
    grid = (N // _TC_BN,)
    return pl.pallas_call(
        _tc_body,
        grid=grid,
        in_specs=[
            pl.BlockSpec((_TC_BN, DIN), lambda i: (i, 0)),
            pl.BlockSpec((DIN, T * 64 + HD), lambda i: (0, 0)),
            pl.BlockSpec((1, HD), lambda i: (0, 0)),
        ],
        out_specs=[
            pl.BlockSpec((T, _TC_BN, HD), lambda i: (0, i, 0)),
            pl.BlockSpec((_TC_BN, HD), lambda i: (i, 0)),
        ],
        out_shape=[
            jax.ShapeDtypeStruct((T, N, HD), jnp.float32),
            jax.ShapeDtypeStruct((N, HD), jnp.float32),
        ],
        compiler_params=pltpu.CompilerParams(
            dimension_semantics=("parallel",)),
    )(feat, wall, bres)


NBUF = 4


def _sc_body(p_hbm, resh_hbm, src_hbm, et_hbm, dst_hbm, ew_hbm, out_hbm,
             src_v, et_v, gidx_v, dst_v, ew_v, rows_v, acc,
             g0, g1, g2, g3, s0, s1, s2, s3):
    gsems = (g0, g1, g2, g3)
    ssems = (s0, s1, s2, s3)
    c = lax.axis_index("c")
    s = lax.axis_index("s")
    w = s * NC + c
    row0 = w * CPW

    pltpu.sync_copy(src_hbm.at[pl.ds(row0, CPW)], src_v)
    pltpu.sync_copy(et_hbm.at[pl.ds(row0, CPW)], et_v)
    pltpu.sync_copy(dst_hbm.at[pl.ds(row0, CPW)], dst_v)
    pltpu.sync_copy(ew_hbm.at[pl.ds(row0 * CH, CPW * CH)], ew_v)

    pltpu.sync_copy(resh_hbm.at[pl.ds(s * ROWS_PT, ROWS_PT)],
                    acc.at[pl.ds(s * ROWS_PT, ROWS_PT)])

    def _gidx_row(r, _):
        for h in range(CH // 16):
            sl = pl.ds(h * 16, 16)
            gidx_v[r, sl] = et_v[r, sl] * N + src_v[r, sl]
        return 0

    lax.fori_loop(0, CPW, _gidx_row, 0)

    plsc.subcore_barrier()

    def _scale(j, b):
        def _group(g, _):
            ew16 = ew_v[pl.ds(j * CH + g * 16, 16)]
            for k16 in range(16):
                k = g * 16 + k16
                wsp = jnp.zeros((16,), jnp.float32) + ew16[k16]
                rows_v[b, k, pl.ds(0, 16)] = rows_v[b, k, pl.ds(0, 16)] * wsp
                rows_v[b, k, pl.ds(16, 16)] = rows_v[b, k, pl.ds(16, 16)] * wsp
            return 0

        lax.fori_loop(0, CH // 16, _group, 0)

    pltpu.async_copy(p_hbm.at[gidx_v.at[0]], rows_v.at[0], gsems[0])
    pltpu.async_copy(p_hbm.at[gidx_v.at[1]], rows_v.at[1], gsems[1])

    def _quad(i, _):
        for b in range(NBUF):
            j = i * NBUF + b
            pltpu.make_async_copy(p_hbm.at[pl.ds(0, CH)], rows_v.at[b],
                                  gsems[b]).wait()
            _scale(j, b)
            b2 = (b + 2) % NBUF
            @pl.when(j >= 2)
            def _():
                pltpu.make_async_copy(rows_v.at[b2], acc.at[pl.ds(0, CH)],
                                      ssems[b2]).wait()

            @pl.when(j + 2 < CPW)
            def _():
                pltpu.async_copy(p_hbm.at[gidx_v.at[j + 2]], rows_v.at[b2],
                                 gsems[b2])

            pltpu.async_copy(rows_v.at[b], acc.at[dst_v.at[j]], ssems[b],
                             add=True)
        return 0

    lax.fori_loop(0, CPW // NBUF, _quad, 0)

    for b in ((CPW - 2) % NBUF, (CPW - 1) % NBUF):
        pltpu.make_async_copy(rows_v.at[b], acc.at[pl.ds(0, CH)],
                              ssems[b]).wait()

    plsc.subcore_barrier()

    pltpu.sync_copy(acc.at[pl.ds(s * ROWS_PT, ROWS_PT)],
                    out_hbm.at[c, pl.ds(s * ROWS_PT, ROWS_PT)])


@functools.cache
def _sc_scatter():
    return pl.kernel(
        _sc_body,
        out_type=jax.ShapeDtypeStruct((NC, NPAD, HD), jnp.float32),
        mesh=plsc.VectorSubcoreMesh(core_axis_name="c", subcore_axis_name="s"),
        scratch_types=[
            pltpu.VMEM((CPW, CH), jnp.int32),
            pltpu.VMEM((CPW, CH), jnp.int32),
            pltpu.VMEM((CPW, CH), jnp.int32),
            pltpu.VMEM((CPW, CH), jnp.int32),
            pltpu.VMEM((CPW * CH,), jnp.float32),
            pltpu.VMEM((NBUF, CH, HD), jnp.float32),
            pltpu.VMEM_SHARED((NPAD, HD), jnp.float32),
        ] + [pltpu.SemaphoreType.DMA] * (2 * NBUF),
        compiler_params=pltpu.CompilerParams(use_tc_tiling_on_sc=False),
    )


def kernel(feat, edge_index, edge_weight, ntype_idxs, etype_idxs,
           W, A_l, A_r, W_res, b_res):
    B = (A_l + A_r).reshape(T, HD, H, D).sum(-1)
    C8 = jnp.tile(jnp.einsum('tik,tkj->tij', W, B), (1, 1, D))
    Wp = W.reshape(T, DIN, H, D).transpose(0, 1, 3, 2).reshape(T, DIN, HD)
    wall = jnp.concatenate(
        [jnp.concatenate([Wp[t], C8[t]], axis=1) for t in range(T)]
        + [W_res], axis=1)

    p_tab, res_half = _tc_precompute(feat, wall, b_res.reshape(1, HD))
    p_flat = p_tab.reshape(T * N, HD)
    res_half = jnp.pad(res_half, ((0, NPAD - N), (0, 0)))

    src = edge_index[0].astype(jnp.int32)
    dst = edge_index[1].astype(jnp.int32)
    et = etype_idxs.astype(jnp.int32)
    ew = edge_weight.astype(jnp.float32)

    pad = EPAD - E
    srcm = jnp.concatenate([src, jnp.zeros((pad,), jnp.int32)]).reshape(-1, CH)
    dstm = jnp.concatenate([dst, jnp.zeros((pad,), jnp.int32)]).reshape(-1, CH)
    etm = jnp.concatenate([et, jnp.zeros((pad,), jnp.int32)]).reshape(-1, CH)
    ewm = jnp.concatenate([ew, jnp.zeros((pad,), jnp.float32)])

    parts = _sc_scatter()(p_flat, res_half, srcm, etm, dstm, ewm)
    return parts[0, :N] + parts[1, :N]


# speedup vs baseline: 183.2269x; 1.1599x over previous
import functools

import jax
import jax.numpy as jnp
import numpy as np
from jax import lax
from jax.experimental import pallas as pl
from jax.experimental.pallas import tpu as pltpu
from jax.experimental.pallas import tpu_sc as plsc

N = 10000
E = 320000
DIN = 128
H = 4
D = 8
HD = H * D
T = 5

NC = 2
NS = 16
CH = 128

CPW0 = 108
CPW1 = 52
NCHUNKS = NS * (CPW0 + CPW1)
ECHUNKS = NCHUNKS + (CPW0 - CPW1)
EPAD = ECHUNKS * CH

NPAD = 10240
ROWS_PT = NPAD // NS
LAST_ROW0 = N - ROWS_PT

NBUF = 4


def _tc_body(feat_ref, wall_ref, bres_ref, p_ref):
    t = pl.program_id(0)
    y = jnp.dot(feat_ref[...], wall_ref[0],
                preferred_element_type=jnp.float32)
    z = y[:, :HD]
    l = y[:, HD:]
    s = jax.nn.sigmoid(jnp.where(l >= 0, l, 0.2 * l))
    p_ref[...] = jnp.where(t == T, (z + bres_ref[...]) * 0.5, z * s)


def _tc_precompute(feat, wall3, bres):
    return pl.pallas_call(
        _tc_body,
        grid=((T + 1),),
        in_specs=[
            pl.BlockSpec((N, DIN), lambda t: (0, 0)),
            pl.BlockSpec((1, DIN, 2 * HD), lambda t: (t, 0, 0)),
            pl.BlockSpec((1, HD), lambda t: (0, 0)),
        ],
        out_specs=pl.BlockSpec((N, HD), lambda t: (t, 0)),
        out_shape=jax.ShapeDtypeStruct(((T + 1) * N, HD), jnp.float32),
        compiler_params=pltpu.CompilerParams(
            dimension_semantics=("arbitrary",)),
    )(feat, wall3, bres)


def _sc_body(p_hbm, src_hbm, et_hbm, dst_hbm, ew_hbm, out_hbm,
             src_v, et_v, gidx_v, dst_v, ew_v, rows_v, acc,
             g0, g1, g2, g3, s0, s1, s2, s3):
    gsems = (g0, g1, g2, g3)
    ssems = (s0, s1, s2, s3)
    c = lax.axis_index("c")
    s = lax.axis_index("s")
    my_cpw = jnp.where(c == 0, CPW0, CPW1)
    row0 = jnp.where(c == 0, s * CPW0, NS * CPW0 + s * CPW1)

    pltpu.sync_copy(src_hbm.at[pl.ds(row0, CPW0)], src_v)
    pltpu.sync_copy(et_hbm.at[pl.ds(row0, CPW0)], et_v)
    pltpu.sync_copy(dst_hbm.at[pl.ds(row0, CPW0)], dst_v)
    pltpu.sync_copy(ew_hbm.at[pl.ds(row0 * CH, CPW0 * CH)], ew_v)

    r0 = jnp.where(s * ROWS_PT > LAST_ROW0, LAST_ROW0, s * ROWS_PT)
    pltpu.sync_copy(p_hbm.at[pl.ds(T * N + r0, ROWS_PT)],
                    acc.at[pl.ds(r0, ROWS_PT)])

    def _gidx_row(r, _):
        for h in range(CH // 16):
            sl = pl.ds(h * 16, 16)
            gidx_v[r, sl] = et_v[r, sl] * N + src_v[r, sl]
        return 0

    lax.fori_loop(0, my_cpw, _gidx_row, 0)

    plsc.subcore_barrier()

    def _scale(j, b):
        def _group(g, _):
            ew16 = ew_v[pl.ds(j * CH + g * 16, 16)]
            for k16 in range(16):
                k = g * 16 + k16
                wsp = jnp.zeros((16,), jnp.float32) + ew16[k16]
                rows_v[b, k, pl.ds(0, 16)] = rows_v[b, k, pl.ds(0, 16)] * wsp
                rows_v[b, k, pl.ds(16, 16)] = rows_v[b, k, pl.ds(16, 16)] * wsp
            return 0

        lax.fori_loop(0, CH // 16, _group, 0)

    pltpu.async_copy(p_hbm.at[gidx_v.at[0]], rows_v.at[0], gsems[0])
    pltpu.async_copy(p_hbm.at[gidx_v.at[1]], rows_v.at[1], gsems[1])

    def _quad(i, _):
        for b in range(NBUF):
            j = i * NBUF + b
            pltpu.make_async_copy(p_hbm.at[pl.ds(0, CH)], rows_v.at[b],
                                  gsems[b]).wait()
            _scale(j, b)
            b2 = (b + 2) % NBUF
            @pl.when(j >= 2)
            def _():
                pltpu.make_async_copy(rows_v.at[b2], acc.at[pl.ds(0, CH)],
                                      ssems[b2]).wait()

            @pl.when(j + 2 < my_cpw)
            def _():
                pltpu.async_copy(p_hbm.at[gidx_v.at[j + 2]], rows_v.at[b2],
                                 gsems[b2])

            pltpu.async_copy(rows_v.at[b], acc.at[dst_v.at[j]], ssems[b],
                             add=True)
        return 0

    lax.fori_loop(0, my_cpw // NBUF, _quad, 0)

    for b in (2, 3):
        pltpu.make_async_copy(rows_v.at[b], acc.at[pl.ds(0, CH)],
                              ssems[b]).wait()

    plsc.subcore_barrier()

    pltpu.sync_copy(acc.at[pl.ds(r0, ROWS_PT)],
                    out_hbm.at[c, pl.ds(r0, ROWS_PT)])


@functools.cache
def _sc_scatter():
    return pl.kernel(
        _sc_body,
        out_type=jax.ShapeDtypeStruct((NC, N, HD), jnp.float32),
        mesh=plsc.VectorSubcoreMesh(core_axis_name="c", subcore_axis_name="s"),
        scratch_types=[
            pltpu.VMEM((CPW0, CH), jnp.int32),
            pltpu.VMEM((CPW0, CH), jnp.int32),
            pltpu.VMEM((CPW0, CH), jnp.int32),
            pltpu.VMEM((CPW0, CH), jnp.int32),
            pltpu.VMEM((CPW0 * CH,), jnp.float32),
            pltpu.VMEM((NBUF, CH, HD), jnp.float32),
            pltpu.VMEM_SHARED((NPAD, HD), jnp.float32),
        ] + [pltpu.SemaphoreType.DMA] * (2 * NBUF),
        compiler_params=pltpu.CompilerParams(use_tc_tiling_on_sc=False),
    )


def _np_consts():
    sd = np.zeros((HD, H), np.float32)
    for k in range(H):
        sd[k * D:(k + 1) * D, k] = 1.0
    pz = np.zeros((HD, 2 * HD), np.float32)
    for k in range(H):
        for d in range(D):
            pz[k * D + d, d * H + k] = 1.0
    pc = np.zeros((H, 2 * HD), np.float32)
    for k in range(H):
        for d in range(D):
            pc[k, HD + d * H + k] = 1.0
    return jnp.asarray(sd), jnp.asarray(pz), jnp.asarray(pc)


def kernel(feat, edge_index, edge_weight, ntype_idxs, etype_idxs,
           W, A_l, A_r, W_res, b_res):
    sd, pz, pc = _np_consts()
    b = jnp.einsum('tij,jk->tik', A_l + A_r, sd)
    m = pz[None] + jnp.einsum('tij,jk->tik', b, pc)
    wall = jnp.einsum('tij,tjk->tik', W, m)
    wres_pad = jnp.pad(W_res, ((0, 0), (0, HD)))[None]
    wall3 = jnp.concatenate([wall, wres_pad], axis=0)

    p6 = _tc_precompute(feat, wall3, b_res.reshape(1, HD))

    src = edge_index[0].astype(jnp.int32)
    dst = edge_index[1].astype(jnp.int32)
    et = etype_idxs.astype(jnp.int32)
    ew = edge_weight.astype(jnp.float32)

    pad = EPAD - E
    srcm = jnp.concatenate([src, jnp.zeros((pad,), jnp.int32)]).reshape(-1, CH)
    dstm = jnp.concatenate([dst, jnp.zeros((pad,), jnp.int32)]).reshape(-1, CH)
    etm = jnp.concatenate([et, jnp.zeros((pad,), jnp.int32)]).reshape(-1, CH)
    ewm = jnp.concatenate([ew, jnp.zeros((pad,), jnp.float32)])

    parts = _sc_scatter()(p6, srcm, etm, dstm, ewm)
    return parts[0] + parts[1]


# speedup vs baseline: 264.4644x; 1.4434x over previous
import functools

import jax
import jax.numpy as jnp
import numpy as np
from jax import lax
from jax.experimental import pallas as pl
from jax.experimental.pallas import tpu as pltpu
from jax.experimental.pallas import tpu_sc as plsc

N = 10000
E = 320000
DIN = 128
H = 4
D = 8
HD = H * D
T = 5

NC = 2
NS = 16
CH = 128
NCH = E // CH

CPW0 = 108
CPW1 = 48
CPW1L = 52
assert NS * CPW0 + (NS - 1) * CPW1 + CPW1L == NCH
WMAX = CPW0

NPAD = 10240
ROWS_PT = NPAD // NS
LAST_ROW0 = N - ROWS_PT

NQT = N // 4

NBUF = 4


NQ = N // 4


def _tc_body(feat_ref, wall_ref, wres_ref, bres_ref, p_ref, res_ref):
    x = feat_ref[...]
    for t in range(T):
        for q in range(4):
            y = jnp.dot(x[q * NQ:(q + 1) * NQ, :], wall_ref[t],
                        preferred_element_type=jnp.float32)
            z = y[:, :HD]
            l = y[:, HD:]
            pv = z * jax.nn.sigmoid(jnp.where(l >= 0, l, 0.2 * l))
            p_ref[pl.ds(t * NQ, NQ), pl.ds(q * HD, HD)] = pv
    y6 = jnp.dot(x, wres_ref[0], preferred_element_type=jnp.float32)
    res_ref[...] = (y6 + bres_ref[...]) * 0.5


def _tc_precompute(feat, wall3, wres, bres):
    return pl.pallas_call(
        _tc_body,
        in_specs=[
            pl.BlockSpec((N, DIN), lambda: (0, 0)),
            pl.BlockSpec((T, DIN, 2 * HD), lambda: (0, 0, 0)),
            pl.BlockSpec((1, DIN, HD), lambda: (0, 0, 0)),
            pl.BlockSpec((1, HD), lambda: (0, 0)),
        ],
        out_specs=[
            pl.BlockSpec((T * NQ, 4 * HD), lambda: (0, 0)),
            pl.BlockSpec((N, HD), lambda: (0, 0)),
        ],
        out_shape=[
            jax.ShapeDtypeStruct((T * NQ, 4 * HD), jnp.float32),
            jax.ShapeDtypeStruct((N, HD), jnp.float32),
        ],
    )(feat, wall3, wres, bres)


def _sc_body(p_hbm, resh_hbm, ei_hbm, et_hbm, ew_hbm, out_hbm,
             src_v, dst_v, et_v, gidx_v, ew_v, rows_v, acc,
             g0, g1, g2, g3, s0, s1, s2, s3):
    gsems = (g0, g1, g2, g3)
    ssems = (s0, s1, s2, s3)
    c = lax.axis_index("c")
    s = lax.axis_index("s")
    my_cpw = jnp.where(c == 0, CPW0,
                       jnp.where(s == NS - 1, CPW1L, CPW1))
    base = jnp.where(c == 0, s * CPW0, NS * CPW0 + s * CPW1)
    start = jnp.minimum(base, NCH - WMAX)
    off = base - start

    pltpu.sync_copy(ei_hbm.at[0, pl.ds(start, WMAX)], src_v)
    pltpu.sync_copy(ei_hbm.at[1, pl.ds(start, WMAX)], dst_v)
    pltpu.sync_copy(et_hbm.at[pl.ds(start, WMAX)], et_v)
    pltpu.sync_copy(ew_hbm.at[pl.ds(start * CH, WMAX * CH)], ew_v)

    r0 = jnp.where(s * ROWS_PT > LAST_ROW0, LAST_ROW0, s * ROWS_PT)
    pltpu.sync_copy(resh_hbm.at[pl.ds(r0, ROWS_PT)],
                    acc.at[pl.ds(r0, ROWS_PT)])

    def _gidx_row(r, _):
        for h in range(CH // 16):
            sl = pl.ds(h * 16, 16)
            s16 = src_v[r, sl]
            one = jnp.ones((16,), jnp.int32)
            zero = jnp.zeros((16,), jnp.int32)
            q = (jnp.where(s16 >= NQT, one, zero)
                 + jnp.where(s16 >= 2 * NQT, one, zero)
                 + jnp.where(s16 >= 3 * NQT, one, zero))
            gidx_v[r, sl] = et_v[r, sl] * N + 4 * s16 - (N - 1) * q
        return 0

    lax.fori_loop(off, off + my_cpw, _gidx_row, 0)

    plsc.subcore_barrier()

    def _scale(j, b):
        def _group(g, _):
            ew16 = ew_v[pl.ds(j * CH + g * 16, 16)]
            for k16 in range(16):
                k = g * 16 + k16
                wsp = jnp.zeros((16,), jnp.float32) + ew16[k16]
                rows_v[b, k, pl.ds(0, 16)] = rows_v[b, k, pl.ds(0, 16)] * wsp
                rows_v[b, k, pl.ds(16, 16)] = rows_v[b, k, pl.ds(16, 16)] * wsp
            return 0

        lax.fori_loop(0, CH // 16, _group, 0)

    pltpu.async_copy(p_hbm.at[gidx_v.at[off]], rows_v.at[0], gsems[0])
    pltpu.async_copy(p_hbm.at[gidx_v.at[off + 1]], rows_v.at[1], gsems[1])

    def _quad(i, _):
        for b in range(NBUF):
            j = i * NBUF + b
            pltpu.make_async_copy(p_hbm.at[pl.ds(0, CH)], rows_v.at[b],
                                  gsems[b]).wait()
            _scale(off + j, b)
            b2 = (b + 2) % NBUF
            @pl.when(j >= 2)
            def _():
                pltpu.make_async_copy(rows_v.at[b2], acc.at[pl.ds(0, CH)],
                                      ssems[b2]).wait()

            @pl.when(j + 2 < my_cpw)
            def _():
                pltpu.async_copy(p_hbm.at[gidx_v.at[off + j + 2]],
                                 rows_v.at[b2], gsems[b2])

            pltpu.async_copy(rows_v.at[b], acc.at[dst_v.at[off + j]],
                             ssems[b], add=True)
        return 0

    lax.fori_loop(0, my_cpw // NBUF, _quad, 0)

    for b in (2, 3):
        pltpu.make_async_copy(rows_v.at[b], acc.at[pl.ds(0, CH)],
                              ssems[b]).wait()

    plsc.subcore_barrier()

    pltpu.sync_copy(acc.at[pl.ds(r0, ROWS_PT)],
                    out_hbm.at[c, pl.ds(r0, ROWS_PT)])


@functools.cache
def _sc_scatter():
    return pl.kernel(
        _sc_body,
        out_type=jax.ShapeDtypeStruct((NC, N, HD), jnp.float32),
        mesh=plsc.VectorSubcoreMesh(core_axis_name="c", subcore_axis_name="s"),
        scratch_types=[
            pltpu.VMEM((WMAX, CH), jnp.int32),
            pltpu.VMEM((WMAX, CH), jnp.int32),
            pltpu.VMEM((WMAX, CH), jnp.int32),
            pltpu.VMEM((WMAX, CH), jnp.int32),
            pltpu.VMEM((WMAX * CH,), jnp.float32),
            pltpu.VMEM((NBUF, CH, HD), jnp.float32),
            pltpu.VMEM_SHARED((NPAD, HD), jnp.float32),
        ] + [pltpu.SemaphoreType.DMA] * (2 * NBUF),
        compiler_params=pltpu.CompilerParams(use_tc_tiling_on_sc=False),
    )


def _np_consts():
    sd = np.zeros((HD, H), np.float32)
    for k in range(H):
        sd[k * D:(k + 1) * D, k] = 1.0
    pz = np.zeros((HD, 2 * HD), np.float32)
    for k in range(H):
        for d in range(D):
            pz[k * D + d, d * H + k] = 1.0
    pc = np.zeros((H, 2 * HD), np.float32)
    for k in range(H):
        for d in range(D):
            pc[k, HD + d * H + k] = 1.0
    return jnp.asarray(sd), jnp.asarray(pz), jnp.asarray(pc)


def kernel(feat, edge_index, edge_weight, ntype_idxs, etype_idxs,
           W, A_l, A_r, W_res, b_res):
    sd, pz, pc = _np_consts()
    b = jnp.einsum('tij,jk->tik', A_l + A_r, sd)
    m = pz[None] + jnp.einsum('tij,jk->tik', b, pc)
    wall3 = jnp.einsum('tij,tjk->tik', W, m)
    wres_pad = W_res[None]

    p5, resh = _tc_precompute(feat, wall3, wres_pad, b_res.reshape(1, HD))
    p5 = p5.reshape(T * N, HD)

    parts = _sc_scatter()(
        p5,
        resh,
        edge_index.astype(jnp.int32).reshape(2, NCH, CH),
        etype_idxs.astype(jnp.int32).reshape(NCH, CH),
        edge_weight.astype(jnp.float32),
    )
    return parts[0] + parts[1]


# speedup vs baseline: 289.2248x; 1.0936x over previous
import functools

import jax
import jax.numpy as jnp
import numpy as np
from jax import lax
from jax.experimental import pallas as pl
from jax.experimental.pallas import tpu as pltpu
from jax.experimental.pallas import tpu_sc as plsc

N = 10000
E = 320000
DIN = 128
H = 4
D = 8
HD = H * D
T = 5

NC = 2
NS = 16
CH = 128
NCH = E // CH

CPW0 = 88
CPW1 = 68
CPW1L = 72
assert NS * CPW0 + (NS - 1) * CPW1 + CPW1L == NCH
WMAX = CPW0

NPAD = 10240
ROWS_PT = NPAD // NS
LAST_ROW0 = N - ROWS_PT

NQT = N // 4

NBUF = 4


NQ = N // 4


def _tc_body(feat_ref, wall_ref, wres_ref, bres_ref, p_ref, res_ref):
    x = feat_ref[...]
    for t in range(T):
        for q in range(4):
            y = jnp.dot(x[q * NQ:(q + 1) * NQ, :], wall_ref[t],
                        preferred_element_type=jnp.float32)
            z = y[:, :HD]
            l = y[:, HD:]
            pv = z * jax.nn.sigmoid(jnp.where(l >= 0, l, 0.2 * l))
            p_ref[pl.ds(t * NQ, NQ), pl.ds(q * HD, HD)] = pv
    y6 = jnp.dot(x, wres_ref[0], preferred_element_type=jnp.float32)
    res_ref[...] = (y6 + bres_ref[...]) * 0.5


def _tc_precompute(feat, wall3, wres, bres):
    return pl.pallas_call(
        _tc_body,
        in_specs=[
            pl.BlockSpec((N, DIN), lambda: (0, 0)),
            pl.BlockSpec((T, DIN, 2 * HD), lambda: (0, 0, 0)),
            pl.BlockSpec((1, DIN, HD), lambda: (0, 0, 0)),
            pl.BlockSpec((1, HD), lambda: (0, 0)),
        ],
        out_specs=[
            pl.BlockSpec((T * NQ, 4 * HD), lambda: (0, 0)),
            pl.BlockSpec((N, HD), lambda: (0, 0)),
        ],
        out_shape=[
            jax.ShapeDtypeStruct((T * NQ, 4 * HD), jnp.float32),
            jax.ShapeDtypeStruct((N, HD), jnp.float32),
        ],
    )(feat, wall3, wres, bres)


def _sc_body(p_hbm, resh_hbm, ei_hbm, et_hbm, ew_hbm, out_hbm,
             src_v, dst_v, et_v, gidx_v, ew_v, rows_v, acc,
             g0, g1, g2, g3, s0, s1, s2, s3):
    gsems = (g0, g1, g2, g3)
    ssems = (s0, s1, s2, s3)
    c = lax.axis_index("c")
    s = lax.axis_index("s")
    my_cpw = jnp.where(c == 0, CPW0,
                       jnp.where(s == NS - 1, CPW1L, CPW1))
    base = jnp.where(c == 0, s * CPW0, NS * CPW0 + s * CPW1)
    start = jnp.minimum(base, NCH - WMAX)
    off = base - start

    pltpu.sync_copy(ei_hbm.at[0, pl.ds(start, WMAX)], src_v)
    pltpu.sync_copy(ei_hbm.at[1, pl.ds(start, WMAX)], dst_v)
    pltpu.sync_copy(et_hbm.at[pl.ds(start, WMAX)], et_v)
    pltpu.sync_copy(ew_hbm.at[pl.ds(start * CH, WMAX * CH)], ew_v)

    r0 = jnp.where(s * ROWS_PT > LAST_ROW0, LAST_ROW0, s * ROWS_PT)
    pltpu.sync_copy(resh_hbm.at[pl.ds(r0, ROWS_PT)],
                    acc.at[pl.ds(r0, ROWS_PT)])

    def _gidx_row(r, _):
        for h in range(CH // 16):
            sl = pl.ds(h * 16, 16)
            s16 = src_v[r, sl]
            one = jnp.ones((16,), jnp.int32)
            zero = jnp.zeros((16,), jnp.int32)
            q = (jnp.where(s16 >= NQT, one, zero)
                 + jnp.where(s16 >= 2 * NQT, one, zero)
                 + jnp.where(s16 >= 3 * NQT, one, zero))
            gidx_v[r, sl] = et_v[r, sl] * N + 4 * s16 - (N - 1) * q
        return 0

    lax.fori_loop(off, off + my_cpw, _gidx_row, 0)

    plsc.subcore_barrier()

    def _scale(j, b):
        def _group(g, _):
            ew16 = ew_v[pl.ds(j * CH + g * 16, 16)]
            for k16 in range(16):
                k = g * 16 + k16
                wsp = jnp.zeros((16,), jnp.float32) + ew16[k16]
                rows_v[b, k, pl.ds(0, 16)] = rows_v[b, k, pl.ds(0, 16)] * wsp
                rows_v[b, k, pl.ds(16, 16)] = rows_v[b, k, pl.ds(16, 16)] * wsp
            return 0

        lax.fori_loop(0, CH // 16, _group, 0)

    pltpu.async_copy(p_hbm.at[gidx_v.at[off]], rows_v.at[0], gsems[0])
    pltpu.async_copy(p_hbm.at[gidx_v.at[off + 1]], rows_v.at[1], gsems[1])

    def _quad(i, _):
        for b in range(NBUF):
            j = i * NBUF + b
            pltpu.make_async_copy(p_hbm.at[pl.ds(0, CH)], rows_v.at[b],
                                  gsems[b]).wait()
            _scale(off + j, b)
            b2 = (b + 2) % NBUF
            @pl.when(j >= 2)
            def _():
                pltpu.make_async_copy(rows_v.at[b2], acc.at[pl.ds(0, CH)],
                                      ssems[b2]).wait()

            @pl.when(j + 2 < my_cpw)
            def _():
                pltpu.async_copy(p_hbm.at[gidx_v.at[off + j + 2]],
                                 rows_v.at[b2], gsems[b2])

            pltpu.async_copy(rows_v.at[b], acc.at[dst_v.at[off + j]],
                             ssems[b], add=True)
        return 0

    lax.fori_loop(0, my_cpw // NBUF, _quad, 0)

    for b in (2, 3):
        pltpu.make_async_copy(rows_v.at[b], acc.at[pl.ds(0, CH)],
                              ssems[b]).wait()

    plsc.subcore_barrier()

    pltpu.sync_copy(acc.at[pl.ds(r0, ROWS_PT)],
                    out_hbm.at[c, pl.ds(r0, ROWS_PT)])


@functools.cache
def _sc_scatter():
    return pl.kernel(
        _sc_body,
        out_type=jax.ShapeDtypeStruct((NC, N, HD), jnp.float32),
        mesh=plsc.VectorSubcoreMesh(core_axis_name="c", subcore_axis_name="s"),
        scratch_types=[
            pltpu.VMEM((WMAX, CH), jnp.int32),
            pltpu.VMEM((WMAX, CH), jnp.int32),
            pltpu.VMEM((WMAX, CH), jnp.int32),
            pltpu.VMEM((WMAX, CH), jnp.int32),
            pltpu.VMEM((WMAX * CH,), jnp.float32),
            pltpu.VMEM((NBUF, CH, HD), jnp.float32),
            pltpu.VMEM_SHARED((NPAD, HD), jnp.float32),
        ] + [pltpu.SemaphoreType.DMA] * (2 * NBUF),
        compiler_params=pltpu.CompilerParams(use_tc_tiling_on_sc=False),
    )


def _np_consts():
    sd = np.zeros((HD, H), np.float32)
    for k in range(H):
        sd[k * D:(k + 1) * D, k] = 1.0
    pz = np.zeros((HD, 2 * HD), np.float32)
    for k in range(H):
        for d in range(D):
            pz[k * D + d, d * H + k] = 1.0
    pc = np.zeros((H, 2 * HD), np.float32)
    for k in range(H):
        for d in range(D):
            pc[k, HD + d * H + k] = 1.0
    return jnp.asarray(sd), jnp.asarray(pz), jnp.asarray(pc)


def kernel(feat, edge_index, edge_weight, ntype_idxs, etype_idxs,
           W, A_l, A_r, W_res, b_res):
    sd, pz, pc = _np_consts()
    b = jnp.einsum('tij,jk->tik', A_l + A_r, sd)
    m = pz[None] + jnp.einsum('tij,jk->tik', b, pc)
    wall3 = jnp.einsum('tij,tjk->tik', W, m)
    wres_pad = W_res[None]

    p5, resh = _tc_precompute(feat, wall3, wres_pad, b_res.reshape(1, HD))
    p5 = p5.reshape(T * N, HD)

    parts = _sc_scatter()(
        p5,
        resh,
        edge_index.astype(jnp.int32).reshape(2, NCH, CH),
        etype_idxs.astype(jnp.int32).reshape(NCH, CH),
        edge_weight.astype(jnp.float32),
    )
    return parts[0] + parts[1]


# speedup vs baseline: 327.3214x; 1.1317x over previous
import functools

import jax
import jax.numpy as jnp
import numpy as np
from jax import lax
from jax.experimental import pallas as pl
from jax.experimental.pallas import tpu as pltpu
from jax.experimental.pallas import tpu_sc as plsc

N = 10000
E = 320000
DIN = 128
H = 4
D = 8
HD = H * D
T = 5

NC = 2
NS = 16
CH = 128
NCH = E // CH

CPW0 = 88
CPW1 = 68
CPW1L = 72
assert NS * CPW0 + (NS - 1) * CPW1 + CPW1L == NCH
WMAX = CPW0

NPAD = 10240
ROWS_PT = NPAD // NS
LAST_ROW0 = N - ROWS_PT

NQT = N // 4

NBUF = 8
LA = 4


NQ = N // 4


def _tc_body(feat_ref, wall_ref, wres_ref, bres_ref, p_ref, res_ref):
    x = feat_ref[...]
    for t in range(T):
        for q in range(4):
            y = jnp.dot(x[q * NQ:(q + 1) * NQ, :], wall_ref[t],
                        preferred_element_type=jnp.float32)
            z = y[:, :HD]
            l = y[:, HD:]
            pv = z * jax.nn.sigmoid(jnp.where(l >= 0, l, 0.2 * l))
            p_ref[pl.ds(t * NQ, NQ), pl.ds(q * HD, HD)] = pv
    y6 = jnp.dot(x, wres_ref[0], preferred_element_type=jnp.float32)
    res_ref[...] = (y6 + bres_ref[...]) * 0.5


def _tc_precompute(feat, wall3, wres, bres):
    return pl.pallas_call(
        _tc_body,
        in_specs=[
            pl.BlockSpec((N, DIN), lambda: (0, 0)),
            pl.BlockSpec((T, DIN, 2 * HD), lambda: (0, 0, 0)),
            pl.BlockSpec((1, DIN, HD), lambda: (0, 0, 0)),
            pl.BlockSpec((1, HD), lambda: (0, 0)),
        ],
        out_specs=[
            pl.BlockSpec((T * NQ, 4 * HD), lambda: (0, 0)),
            pl.BlockSpec((N, HD), lambda: (0, 0)),
        ],
        out_shape=[
            jax.ShapeDtypeStruct((T * NQ, 4 * HD), jnp.float32),
            jax.ShapeDtypeStruct((N, HD), jnp.float32),
        ],
    )(feat, wall3, wres, bres)


def _sc_body(p_hbm, resh_hbm, ei_hbm, et_hbm, ew_hbm, out_hbm,
             src_v, dst_v, et_v, gidx_v, ew_v, rows_v, acc,
             g0, g1, g2, g3, s0, s1, s2, s3,
             g4, g5, g6, g7, s4, s5, s6, s7):
    gsems = (g0, g1, g2, g3, g4, g5, g6, g7)
    ssems = (s0, s1, s2, s3, s4, s5, s6, s7)
    c = lax.axis_index("c")
    s = lax.axis_index("s")
    my_cpw = jnp.where(c == 0, CPW0,
                       jnp.where(s == NS - 1, CPW1L, CPW1))
    base = jnp.where(c == 0, s * CPW0, NS * CPW0 + s * CPW1)
    start = jnp.minimum(base, NCH - WMAX)
    off = base - start

    pltpu.sync_copy(ei_hbm.at[0, pl.ds(start, WMAX)], src_v)
    pltpu.sync_copy(ei_hbm.at[1, pl.ds(start, WMAX)], dst_v)
    pltpu.sync_copy(et_hbm.at[pl.ds(start, WMAX)], et_v)
    pltpu.sync_copy(ew_hbm.at[pl.ds(start * CH, WMAX * CH)], ew_v)

    r0 = jnp.where(s * ROWS_PT > LAST_ROW0, LAST_ROW0, s * ROWS_PT)
    pltpu.sync_copy(resh_hbm.at[pl.ds(r0, ROWS_PT)],
                    acc.at[pl.ds(r0, ROWS_PT)])


    def _gidx_row(r, _):
        for h in range(CH // 16):
            sl = pl.ds(h * 16, 16)
            s16 = src_v[r, sl]
            one = jnp.ones((16,), jnp.int32)
            zero = jnp.zeros((16,), jnp.int32)
            q = (jnp.where(s16 >= NQT, one, zero)
                 + jnp.where(s16 >= 2 * NQT, one, zero)
                 + jnp.where(s16 >= 3 * NQT, one, zero))
            gidx_v[r, sl] = et_v[r, sl] * N + 4 * s16 - (N - 1) * q
        return 0

    lax.fori_loop(off, off + my_cpw, _gidx_row, 0)

    plsc.subcore_barrier()

    def _scale(j, b):
        def _group(g, _):
            ew16 = ew_v[pl.ds(j * CH + g * 16, 16)]
            for k16 in range(16):
                k = g * 16 + k16
                wsp = jnp.zeros((16,), jnp.float32) + ew16[k16]
                rows_v[b, k, pl.ds(0, 16)] = rows_v[b, k, pl.ds(0, 16)] * wsp
                rows_v[b, k, pl.ds(16, 16)] = rows_v[b, k, pl.ds(16, 16)] * wsp
            return 0

        lax.fori_loop(0, CH // 16, _group, 0)

    for b in range(LA):
        pltpu.async_copy(p_hbm.at[gidx_v.at[off + b]], rows_v.at[b],
                         gsems[b])

    def _step(j, b):
        pltpu.make_async_copy(p_hbm.at[pl.ds(0, CH)], rows_v.at[b],
                              gsems[b]).wait()
        _scale(off + j, b)
        b2 = (b + LA) % NBUF
        @pl.when(j >= LA)
        def _():
            pltpu.make_async_copy(rows_v.at[b2], acc.at[pl.ds(0, CH)],
                                  ssems[b2]).wait()

        @pl.when(j + LA < my_cpw)
        def _():
            pltpu.async_copy(p_hbm.at[gidx_v.at[off + j + LA]],
                             rows_v.at[b2], gsems[b2])

        pltpu.async_copy(rows_v.at[b], acc.at[dst_v.at[off + j]],
                         ssems[b], add=True)

    def _oct(i, _):
        for b in range(NBUF):
            _step(i * NBUF + b, b)
        return 0

    lax.fori_loop(0, my_cpw // NBUF, _oct, 0)

    @pl.when(my_cpw % NBUF == 4)
    def _():
        for b in range(4):
            _step(my_cpw - 4 + b, b)

    @pl.when(my_cpw % NBUF == 4)
    def _():
        for b in (0, 1, 2, 3):
            pltpu.make_async_copy(rows_v.at[b], acc.at[pl.ds(0, CH)],
                                  ssems[b]).wait()

    @pl.when(my_cpw % NBUF == 0)
    def _():
        for b in (4, 5, 6, 7):
            pltpu.make_async_copy(rows_v.at[b], acc.at[pl.ds(0, CH)],
                                  ssems[b]).wait()

    plsc.subcore_barrier()

    pltpu.sync_copy(acc.at[pl.ds(r0, ROWS_PT)],
                    out_hbm.at[c, pl.ds(r0, ROWS_PT)])


@functools.cache
def _sc_scatter():
    return pl.kernel(
        _sc_body,
        out_type=jax.ShapeDtypeStruct((NC, N, HD), jnp.float32),
        mesh=plsc.VectorSubcoreMesh(core_axis_name="c", subcore_axis_name="s"),
        scratch_types=[
            pltpu.VMEM((WMAX, CH), jnp.int32),
            pltpu.VMEM((WMAX, CH), jnp.int32),
            pltpu.VMEM((WMAX, CH), jnp.int32),
            pltpu.VMEM((WMAX, CH), jnp.int32),
            pltpu.VMEM((WMAX * CH,), jnp.float32),
            pltpu.VMEM((NBUF, CH, HD), jnp.float32),
            pltpu.VMEM_SHARED((NPAD, HD), jnp.float32),
        ] + [pltpu.SemaphoreType.DMA] * (2 * NBUF),
        compiler_params=pltpu.CompilerParams(use_tc_tiling_on_sc=False),
    )


def _np_consts():
    sd = np.zeros((HD, H), np.float32)
    for k in range(H):
        sd[k * D:(k + 1) * D, k] = 1.0
    pz = np.zeros((HD, 2 * HD), np.float32)
    for k in range(H):
        for d in range(D):
            pz[k * D + d, d * H + k] = 1.0
    pc = np.zeros((H, 2 * HD), np.float32)
    for k in range(H):
        for d in range(D):
            pc[k, HD + d * H + k] = 1.0
    return jnp.asarray(sd), jnp.asarray(pz), jnp.asarray(pc)


def kernel(feat, edge_index, edge_weight, ntype_idxs, etype_idxs,
           W, A_l, A_r, W_res, b_res):
    sd, pz, pc = _np_consts()
    b = jnp.einsum('tij,jk->tik', A_l + A_r, sd)
    m = pz[None] + jnp.einsum('tij,jk->tik', b, pc)
    wall3 = jnp.einsum('tij,tjk->tik', W, m)
    wres_pad = W_res[None]

    p5, resh = _tc_precompute(feat, wall3, wres_pad, b_res.reshape(1, HD))
    p5 = p5.reshape(T * N, HD)

    parts = _sc_scatter()(
        p5,
        resh,
        edge_index.astype(jnp.int32).reshape(2, NCH, CH),
        etype_idxs.astype(jnp.int32).reshape(NCH, CH),
        edge_weight.astype(jnp.float32),
    )
    return parts[0] + parts[1]


# speedup vs baseline: 332.3168x; 1.0153x over previous
import functools

import jax
import jax.numpy as jnp
import numpy as np
from jax import lax
from jax.experimental import pallas as pl
from jax.experimental.pallas import tpu as pltpu
from jax.experimental.pallas import tpu_sc as plsc

N = 10000
E = 320000
DIN = 128
H = 4
D = 8
HD = H * D
T = 5

NC = 2
NS = 16
CH = 128
NCH = E // CH

CPW0 = 84
CPW1 = 72
CPW1L = 76
assert NS * CPW0 + (NS - 1) * CPW1 + CPW1L == NCH
WMAX = CPW0

NPAD = 10240
ROWS_PT = NPAD // NS
LAST_ROW0 = N - ROWS_PT

NQT = N // 4

NBUF = 8
LA = 4


NQ = N // 4


def _tc_body(feat_ref, wall_ref, wres_ref, bres_ref, p_ref, res_ref):
    x = feat_ref[...]
    for t in range(T):
        for q in range(4):
            y = jnp.dot(x[q * NQ:(q + 1) * NQ, :], wall_ref[t],
                        preferred_element_type=jnp.float32)
            z = y[:, :HD]
            l = y[:, HD:]
            pv = z * jax.nn.sigmoid(jnp.where(l >= 0, l, 0.2 * l))
            p_ref[pl.ds(t * NQ, NQ), pl.ds(q * HD, HD)] = pv
    y6 = jnp.dot(x, wres_ref[0], preferred_element_type=jnp.float32)
    res_ref[...] = (y6 + bres_ref[...]) * 0.5


def _tc_precompute(feat, wall3, wres, bres):
    return pl.pallas_call(
        _tc_body,
        in_specs=[
            pl.BlockSpec((N, DIN), lambda: (0, 0)),
            pl.BlockSpec((T, DIN, 2 * HD), lambda: (0, 0, 0)),
            pl.BlockSpec((1, DIN, HD), lambda: (0, 0, 0)),
            pl.BlockSpec((1, HD), lambda: (0, 0)),
        ],
        out_specs=[
            pl.BlockSpec((T * NQ, 4 * HD), lambda: (0, 0)),
            pl.BlockSpec((N, HD), lambda: (0, 0)),
        ],
        out_shape=[
            jax.ShapeDtypeStruct((T * NQ, 4 * HD), jnp.float32),
            jax.ShapeDtypeStruct((N, HD), jnp.float32),
        ],
    )(feat, wall3, wres, bres)


def _sc_body(p_hbm, resh_hbm, ei_hbm, et_hbm, ew_hbm, out_hbm,
             src_v, dst_v, et_v, gidx_v, ew_v, rows_v, acc,
             g0, g1, g2, g3, s0, s1, s2, s3,
             g4, g5, g6, g7, s4, s5, s6, s7):
    gsems = (g0, g1, g2, g3, g4, g5, g6, g7)
    ssems = (s0, s1, s2, s3, s4, s5, s6, s7)
    c = lax.axis_index("c")
    s = lax.axis_index("s")
    my_cpw = jnp.where(c == 0, CPW0,
                       jnp.where(s == NS - 1, CPW1L, CPW1))
    base = jnp.where(c == 0, s * CPW0, NS * CPW0 + s * CPW1)
    start = jnp.minimum(base, NCH - WMAX)
    off = base - start

    pltpu.sync_copy(ei_hbm.at[0, pl.ds(start, WMAX)], src_v)
    pltpu.sync_copy(ei_hbm.at[1, pl.ds(start, WMAX)], dst_v)
    pltpu.sync_copy(et_hbm.at[pl.ds(start, WMAX)], et_v)
    pltpu.sync_copy(ew_hbm.at[pl.ds(start * CH, WMAX * CH)], ew_v)

    r0 = jnp.where(s * ROWS_PT > LAST_ROW0, LAST_ROW0, s * ROWS_PT)
    pltpu.sync_copy(resh_hbm.at[pl.ds(r0, ROWS_PT)],
                    acc.at[pl.ds(r0, ROWS_PT)])


    def _gidx_row(r, _):
        for h in range(CH // 16):
            sl = pl.ds(h * 16, 16)
            s16 = src_v[r, sl]
            one = jnp.ones((16,), jnp.int32)
            zero = jnp.zeros((16,), jnp.int32)
            q = (jnp.where(s16 >= NQT, one, zero)
                 + jnp.where(s16 >= 2 * NQT, one, zero)
                 + jnp.where(s16 >= 3 * NQT, one, zero))
            gidx_v[r, sl] = et_v[r, sl] * N + 4 * s16 - (N - 1) * q
        return 0

    lax.fori_loop(off, off + my_cpw, _gidx_row, 0)

    plsc.subcore_barrier()

    def _scale(j, b):
        def _group(g, _):
            ew16 = ew_v[pl.ds(j * CH + g * 16, 16)]
            for k16 in range(16):
                k = g * 16 + k16
                wsp = jnp.zeros((16,), jnp.float32) + ew16[k16]
                rows_v[b, k, pl.ds(0, 16)] = rows_v[b, k, pl.ds(0, 16)] * wsp
                rows_v[b, k, pl.ds(16, 16)] = rows_v[b, k, pl.ds(16, 16)] * wsp
            return 0

        lax.fori_loop(0, CH // 16, _group, 0)

    for b in range(LA):
        pltpu.async_copy(p_hbm.at[gidx_v.at[off + b]], rows_v.at[b],
                         gsems[b])

    def _step(j, b):
        pltpu.make_async_copy(p_hbm.at[pl.ds(0, CH)], rows_v.at[b],
                              gsems[b]).wait()
        _scale(off + j, b)
        b2 = (b + LA) % NBUF
        @pl.when(j >= LA)
        def _():
            pltpu.make_async_copy(rows_v.at[b2], acc.at[pl.ds(0, CH)],
                                  ssems[b2]).wait()

        @pl.when(j + LA < my_cpw)
        def _():
            pltpu.async_copy(p_hbm.at[gidx_v.at[off + j + LA]],
                             rows_v.at[b2], gsems[b2])

        pltpu.async_copy(rows_v.at[b], acc.at[dst_v.at[off + j]],
                         ssems[b], add=True)

    def _oct(i, _):
        for b in range(NBUF):
            _step(i * NBUF + b, b)
        return 0

    lax.fori_loop(0, my_cpw // NBUF, _oct, 0)

    @pl.when(my_cpw % NBUF == 4)
    def _():
        for b in range(4):
            _step(my_cpw - 4 + b, b)

    @pl.when(my_cpw % NBUF == 4)
    def _():
        for b in (0, 1, 2, 3):
            pltpu.make_async_copy(rows_v.at[b], acc.at[pl.ds(0, CH)],
                                  ssems[b]).wait()

    @pl.when(my_cpw % NBUF == 0)
    def _():
        for b in (4, 5, 6, 7):
            pltpu.make_async_copy(rows_v.at[b], acc.at[pl.ds(0, CH)],
                                  ssems[b]).wait()

    plsc.subcore_barrier()

    pltpu.sync_copy(acc.at[pl.ds(r0, ROWS_PT)],
                    out_hbm.at[c, pl.ds(r0, ROWS_PT)])


@functools.cache
def _sc_scatter():
    return pl.kernel(
        _sc_body,
        out_type=jax.ShapeDtypeStruct((NC, N, HD), jnp.float32),
        mesh=plsc.VectorSubcoreMesh(core_axis_name="c", subcore_axis_name="s"),
        scratch_types=[
            pltpu.VMEM((WMAX, CH), jnp.int32),
            pltpu.VMEM((WMAX, CH), jnp.int32),
            pltpu.VMEM((WMAX, CH), jnp.int32),
            pltpu.VMEM((WMAX, CH), jnp.int32),
            pltpu.VMEM((WMAX * CH,), jnp.float32),
            pltpu.VMEM((NBUF, CH, HD), jnp.float32),
            pltpu.VMEM_SHARED((NPAD, HD), jnp.float32),
        ] + [pltpu.SemaphoreType.DMA] * (2 * NBUF),
        compiler_params=pltpu.CompilerParams(use_tc_tiling_on_sc=False),
    )


def _np_consts():
    sd = np.zeros((HD, H), np.float32)
    for k in range(H):
        sd[k * D:(k + 1) * D, k] = 1.0
    pz = np.zeros((HD, 2 * HD), np.float32)
    for k in range(H):
        for d in range(D):
            pz[k * D + d, d * H + k] = 1.0
    pc = np.zeros((H, 2 * HD), np.float32)
    for k in range(H):
        for d in range(D):
            pc[k, HD + d * H + k] = 1.0
    return jnp.asarray(sd), jnp.asarray(pz), jnp.asarray(pc)


def kernel(feat, edge_index, edge_weight, ntype_idxs, etype_idxs,
           W, A_l, A_r, W_res, b_res):
    sd, pz, pc = _np_consts()
    b = jnp.einsum('tij,jk->tik', A_l + A_r, sd)
    m = pz[None] + jnp.einsum('tij,jk->tik', b, pc)
    wall3 = jnp.einsum('tij,tjk->tik', W, m)
    wres_pad = W_res[None]

    p5, resh = _tc_precompute(feat, wall3, wres_pad, b_res.reshape(1, HD))
    p5 = p5.reshape(T * N, HD)

    parts = _sc_scatter()(
        p5,
        resh,
        edge_index.astype(jnp.int32).reshape(2, NCH, CH),
        etype_idxs.astype(jnp.int32).reshape(NCH, CH),
        edge_weight.astype(jnp.float32),
    )
    return parts[0] + parts[1]
